# baseline (speedup 1.0000x reference)
import jax
import jax.numpy as jnp
from jax.experimental import pallas as pl
from jax.experimental.pallas import tpu as pltpu


_LOG2E = 1.4426950408889634
_BN = 512


def _attn_kernel(self_ref, feats_ref, neigh_hbm, a_ref, out_ref,
                 nbuf_ref, f1_ref, f2_ref, fb_ref, sem_ref):
    n, d = self_ref.shape
    m = feats_ref.shape[0]
    nsteps = n // _BN

    a2 = a_ref[d:, :]
    vt = (feats_ref[...] @ (a2 * _LOG2E)).T
    f1_ref[...] = jnp.exp2(vt).astype(jnp.bfloat16)
    f2_ref[...] = jnp.exp2(0.2 * vt).astype(jnp.bfloat16)
    col = jax.lax.broadcasted_iota(jnp.int32, (m, 128), 1)
    fb_ref[:, :128] = jnp.where(col == 0, 1.0, 0.0).astype(jnp.bfloat16)
    fb_ref[:, 128:] = feats_ref[...].astype(jnp.bfloat16)

    a1 = a_ref[:d, :]

    def copy_in(i, slot):
        return pltpu.make_async_copy(
            neigh_hbm.at[pl.ds(i * _BN, _BN), :],
            nbuf_ref.at[slot],
            sem_ref.at[slot],
        )

    copy_in(0, 0).start()

    def step(i, _):
        slot = jax.lax.rem(i, 2)

        @pl.when(i + 1 < nsteps)
        def _():
            copy_in(i + 1, 1 - slot).start()

        copy_in(i, slot).wait()

        u = self_ref[pl.ds(i * _BN, _BN), :] @ (a1 * _LOG2E)
        e1 = jnp.exp2(u).astype(jnp.bfloat16)
        e2 = jnp.exp2(0.2 * u).astype(jnp.bfloat16)
        mk = nbuf_ref[slot].astype(jnp.bfloat16)
        p = jnp.maximum(e1 * f1_ref[...], e2 * f2_ref[...]) * mk
        o = jnp.dot(p, fb_ref[...],
                    preferred_element_type=jnp.float32)
        l = o[:, 0:1]
        out_ref[pl.ds(i * _BN, _BN), :] = (
            o[:, 128:] * (1.0 / jnp.where(l == 0.0, 1.0, l)))
        return 0

    jax.lax.fori_loop(0, nsteps, step, 0)


@jax.jit
def kernel(self_feats, features_neighs, neigh_matrix, a):
    n, d = self_feats.shape
    m = features_neighs.shape[0]
    return pl.pallas_call(
        _attn_kernel,
        in_specs=[
            pl.BlockSpec(memory_space=pltpu.MemorySpace.VMEM),
            pl.BlockSpec(memory_space=pltpu.MemorySpace.VMEM),
            pl.BlockSpec(memory_space=pltpu.MemorySpace.HBM),
            pl.BlockSpec(memory_space=pltpu.MemorySpace.VMEM),
        ],
        out_specs=pl.BlockSpec(memory_space=pltpu.MemorySpace.VMEM),
        out_shape=jax.ShapeDtypeStruct((n, d), jnp.float32),
        scratch_shapes=[
            pltpu.VMEM((2, _BN, m), jnp.int32),
            pltpu.VMEM((1, m), jnp.bfloat16),
            pltpu.VMEM((1, m), jnp.bfloat16),
            pltpu.VMEM((m, 128 + d), jnp.bfloat16),
            pltpu.SemaphoreType.DMA((2,)),
        ],
    )(self_feats, features_neighs, neigh_matrix, a)


# speedup vs baseline: 1.0204x; 1.0204x over previous
import jax
import jax.numpy as jnp
from jax.experimental import pallas as pl
from jax.experimental.pallas import tpu as pltpu


_LOG2E = 1.4426950408889634


def _attn_kernel(self_ref, feats_ref, neigh_ref, a_ref, out_ref,
                 f1_ref, f2_ref, fb_ref):
    d = self_ref.shape[1]

    @pl.when(pl.program_id(0) == 0)
    def _():
        a2 = a_ref[d:, :]
        vt = (feats_ref[...] @ (a2 * _LOG2E)).T
        f1_ref[...] = jnp.exp2(vt).astype(jnp.bfloat16)
        f2_ref[...] = jnp.exp2(0.2 * vt).astype(jnp.bfloat16)
        m = feats_ref.shape[0]
        col = jax.lax.broadcasted_iota(jnp.int32, (m, 128), 1)
        fb_ref[:, :128] = jnp.where(col == 0, 1.0, 0.0).astype(jnp.bfloat16)
        fb_ref[:, 128:] = feats_ref[...].astype(jnp.bfloat16)

    a1 = a_ref[:d, :]
    u = self_ref[...] @ (a1 * _LOG2E)
    e1 = jnp.exp2(u).astype(jnp.bfloat16)
    e2 = jnp.exp2(0.2 * u).astype(jnp.bfloat16)
    mk = neigh_ref[...].astype(jnp.bfloat16)
    p = jnp.maximum(e1 * f1_ref[...], e2 * f2_ref[...]) * mk
    o = jnp.dot(p, fb_ref[...],
                preferred_element_type=jnp.float32)
    l = o[:, 0:1]
    out_ref[...] = o[:, 128:] * (1.0 / jnp.where(l == 0.0, 1.0, l))


@jax.jit
def kernel(self_feats, features_neighs, neigh_matrix, a):
    n, d = self_feats.shape
    m = features_neighs.shape[0]
    bn = 512
    grid = (n // bn,)
    return pl.pallas_call(
        _attn_kernel,
        grid=grid,
        in_specs=[
            pl.BlockSpec((bn, d), lambda i: (i, 0)),
            pl.BlockSpec((m, d), lambda i: (0, 0)),
            pl.BlockSpec((bn, m), lambda i: (i, 0)),
            pl.BlockSpec((2 * d, 1), lambda i: (0, 0)),
        ],
        out_specs=pl.BlockSpec((bn, d), lambda i: (i, 0)),
        out_shape=jax.ShapeDtypeStruct((n, d), jnp.float32),
        scratch_shapes=[pltpu.VMEM((1, m), jnp.bfloat16),
                        pltpu.VMEM((1, m), jnp.bfloat16),
                        pltpu.VMEM((m, 128 + d), jnp.bfloat16)],
        compiler_params=pltpu.CompilerParams(
            dimension_semantics=("parallel",),
        ),
    )(self_feats, features_neighs, neigh_matrix, a)
